# baseline (speedup 1.0000x reference)
import functools

import jax
import jax.numpy as jnp
from jax import lax
from jax.experimental import pallas as pl
from jax.experimental.pallas import tpu as pltpu
from jax.experimental.pallas import tpu_sc as plsc

_F32 = jnp.float32
_BF16 = jnp.bfloat16
_I32 = jnp.int32
_HMASK = -65536


_N = 10000
_NP = 10240
_E = 320000
_ER = 2560
_NW = 32
_RW = _ER // _NW
_L = 50000
_LR = 512
_LRW = _LR // _NW
_STRIPE = _NP // 16


def _mesh():
    return plsc.VectorSubcoreMesh(core_axis_name="c", subcore_axis_name="s")


def _dense1_call(xu, xi, W1_pu, b1_pu, W1_pi, b1_pi, ls1, ld1, Wpu1, bpu1):
    R = 2048
    G = _NP // R

    def body(xu_ref, xi_ref, wpu_ref, bpu_ref, wpi_ref, bpi_ref, ls_ref,
             ld_ref, wu_ref, bu_ref,
             pu_ref, as_ref, ad_ref, u1_ref, scal_ref):
        i = pl.program_id(0)
        xub = xu_ref[...]
        xib = xi_ref[...]
        pu = jnp.dot(xub, wpu_ref[...], preferred_element_type=_F32)
        pu = pu + bpu_ref[...][None, :]
        pu_ref[...] = pu
        asb = jnp.sum(pu * ls_ref[...][None, :], axis=1)
        as_ref[...] = asb
        pi = jnp.dot(xib, wpi_ref[...], preferred_element_type=_F32)
        pi = pi + bpi_ref[...][None, :]
        adb = jnp.sum(pi * ld_ref[...][None, :], axis=1)
        ad_ref[...] = adb
        u1 = jnp.dot(xub, wu_ref[...], preferred_element_type=_F32)
        u1_ref[...] = jnp.maximum(u1 + bu_ref[...][None, :], 0.0)
        row = jnp.concatenate([
            jnp.full((1, 128), jnp.max(asb), _F32),
            jnp.full((1, 128), jnp.max(adb), _F32),
            jnp.full((6, 128), -1e30, _F32),
        ], axis=0)

        @pl.when(i == 0)
        def _():
            scal_ref[...] = row

        @pl.when(i != 0)
        def _():
            scal_ref[...] = jnp.maximum(scal_ref[...], row)

    def fullspec(s):
        return pl.BlockSpec(s, lambda i: tuple(0 for _ in s))

    return pl.pallas_call(
        body,
        grid=(G,),
        in_specs=[
            pl.BlockSpec((R, 128), lambda i: (i, 0)),
            pl.BlockSpec((R, 128), lambda i: (i, 0)),
            fullspec((128, 128)), fullspec((128,)),
            fullspec((128, 128)), fullspec((128,)),
            fullspec((128,)), fullspec((128,)),
            fullspec((128, 128)), fullspec((128,)),
        ],
        out_specs=[
            pl.BlockSpec((R, 128), lambda i: (i, 0)),
            pl.BlockSpec((R,), lambda i: (i,)),
            pl.BlockSpec((R,), lambda i: (i,)),
            pl.BlockSpec((R, 128), lambda i: (i, 0)),
            pl.BlockSpec((8, 128), lambda i: (0, 0)),
        ],
        out_shape=[
            jax.ShapeDtypeStruct((_NP, 128), _F32),
            jax.ShapeDtypeStruct((_NP,), _F32),
            jax.ShapeDtypeStruct((_NP,), _F32),
            jax.ShapeDtypeStruct((_NP, 128), _F32),
            jax.ShapeDtypeStruct((8, 128), _F32),
        ],
    )(xu, xi, W1_pu, b1_pu, W1_pi, b1_pi, ls1, ld1, Wpu1, bpu1)


def _dense2_call(m0, m1, user1, W2_pu, b2_pu, W2_pi, b2_pi, ls2, ld2,
                 Wpu2, bpu2, Wpost, bpost):
    R = 2048
    G = _NP // R

    def body(m0_ref, m1_ref, u1_ref, wpu_ref, bpu_ref, wpi_ref, bpi_ref,
             ls_ref, ld_ref, wu_ref, bu_ref, wpost_ref, bpost_ref,
             pu2_ref, as_ref, ad_ref, u2_ref, scal_ref, wvb_ref):
        i = pl.program_id(0)
        it1 = jnp.maximum(m0_ref[...] + m1_ref[...], 0.0)
        u1 = u1_ref[...]
        pu2 = jnp.dot(u1, wpu_ref[...], preferred_element_type=_F32)
        pu2 = pu2 + bpu_ref[...][None, :]
        pu2_ref[...] = pu2
        asb = jnp.sum(pu2 * ls_ref[...][None, :], axis=1)
        as_ref[...] = asb
        pi2 = jnp.dot(it1, wpi_ref[...], preferred_element_type=_F32)
        pi2 = pi2 + bpi_ref[...][None, :]
        adb = jnp.sum(pi2 * ld_ref[...][None, :], axis=1)
        ad_ref[...] = adb
        wv = jnp.sum(wpost_ref[...], axis=1)
        b0 = jnp.sum(bpost_ref[...])
        u2 = jnp.dot(u1, wu_ref[...], preferred_element_type=_F32)
        u2 = jnp.maximum(u2 + bu_ref[...][None, :], 0.0)
        u2_ref[...] = u2 * wv[None, :]
        wvb_ref[...] = jnp.concatenate([wv, jnp.full((64,), b0, _F32)], axis=0)
        row = jnp.concatenate([
            jnp.full((1, 128), jnp.max(asb), _F32),
            jnp.full((1, 128), jnp.max(adb), _F32),
            jnp.full((6, 128), -1e30, _F32),
        ], axis=0)

        @pl.when(i == 0)
        def _():
            scal_ref[...] = row

        @pl.when(i != 0)
        def _():
            scal_ref[...] = jnp.maximum(scal_ref[...], row)

    def fullspec(s):
        return pl.BlockSpec(s, lambda i: tuple(0 for _ in s))

    return pl.pallas_call(
        body,
        grid=(G,),
        in_specs=[
            pl.BlockSpec((R, 128), lambda i: (i, 0)),
            pl.BlockSpec((R, 128), lambda i: (i, 0)),
            pl.BlockSpec((R, 128), lambda i: (i, 0)),
            fullspec((128, 64)), fullspec((64,)),
            fullspec((128, 64)), fullspec((64,)),
            fullspec((64,)), fullspec((64,)),
            fullspec((128, 64)), fullspec((64,)),
            fullspec((64, 2)), fullspec((2,)),
        ],
        out_specs=[
            pl.BlockSpec((R, 64), lambda i: (i, 0)),
            pl.BlockSpec((R,), lambda i: (i,)),
            pl.BlockSpec((R,), lambda i: (i,)),
            pl.BlockSpec((R, 64), lambda i: (i, 0)),
            pl.BlockSpec((8, 128), lambda i: (0, 0)),
            pl.BlockSpec((128,), lambda i: (0,)),
        ],
        out_shape=[
            jax.ShapeDtypeStruct((_NP, 64), _F32),
            jax.ShapeDtypeStruct((_NP,), _F32),
            jax.ShapeDtypeStruct((_NP,), _F32),
            jax.ShapeDtypeStruct((_NP, 64), _F32),
            jax.ShapeDtypeStruct((8, 128), _F32),
            jax.ShapeDtypeStruct((128,), _F32),
        ],
    )(m0, m1, user1, W2_pu, b2_pu, W2_pi, b2_pi, ls2, ld2, Wpu2, bpu2,
      Wpost, bpost)


def _make_pass1():
    @functools.partial(
        pl.kernel,
        out_type=(
            jax.ShapeDtypeStruct((_ER, 128), _F32),
            jax.ShapeDtypeStruct((_NP,), _F32),
            jax.ShapeDtypeStruct((_NP,), _F32),
        ),
        mesh=_mesh(),
        compiler_params=pltpu.CompilerParams(use_tc_tiling_on_sc=False),
        scratch_types=(
            pltpu.VMEM((4, 128), _I32),
            pltpu.VMEM((4, 128), _I32),
            pltpu.VMEM((4, 128), _F32),
            pltpu.VMEM((128,), _F32),
            pltpu.VMEM((128,), _F32),
            pltpu.VMEM((128,), _F32),
            pltpu.VMEM((128,), _F32),
            pltpu.VMEM((8, 128), _F32),
            pltpu.VMEM((_STRIPE,), _F32),
            pltpu.VMEM_SHARED((_NP,), _F32),
            pltpu.SemaphoreType.DMA,
            pltpu.SemaphoreType.DMA,
        ),
    )
    def k(src_hbm, dst_hbm, as_hbm, ad_hbm, scal_hbm,
          e_hbm, s0_hbm, s1_hbm,
          idx_s, idx_d, ev, asv0, adv0, asv1, adv1, scalv, zbuf, s_sh,
          sem0, sem1):
        cid = lax.axis_index("c")
        sid = lax.axis_index("s")
        wid = sid * 2 + cid

        def zinit(i, _):
            zbuf[pl.ds(i * 16, 16)] = jnp.zeros((16,), _F32)
            return 0

        lax.fori_loop(0, _STRIPE // 16, zinit, 0)
        pltpu.sync_copy(zbuf, s_sh.at[pl.ds(sid * _STRIPE, _STRIPE)])
        plsc.subcore_barrier()

        base = wid * _RW
        pltpu.sync_copy(scal_hbm, scalv)
        cpre = scalv[0, pl.ds(0, 16)] + scalv[1, pl.ds(0, 16)]
        cmaxv = jnp.where(cpre >= 0.0, cpre, 0.2 * cpre)
        bufs = [(asv0, adv0, sem0), (asv1, adv1, sem1)]

        def fire(j, asv, adv, sem):
            pltpu.async_copy(as_hbm.at[idx_s.at[j]], asv, sem)
            pltpu.async_copy(ad_hbm.at[idx_d.at[j]], adv, sem)

        def drain(j, asv, adv, sem):
            pltpu.make_async_copy(as_hbm.at[idx_s.at[j]], asv, sem).wait()
            pltpu.make_async_copy(ad_hbm.at[idx_d.at[j]], adv, sem).wait()

        def group(g, _):
            gbase = base + g * 4
            pltpu.sync_copy(src_hbm.at[pl.ds(gbase, 4)], idx_s)
            pltpu.sync_copy(dst_hbm.at[pl.ds(gbase, 4)], idx_d)
            fire(0, *bufs[0])
            for j in range(4):
                asv, adv, sem = bufs[j % 2]
                drain(j, asv, adv, sem)
                if j < 3:
                    fire(j + 1, *bufs[(j + 1) % 2])
                for t in range(8):
                    sl = pl.ds(t * 16, 16)
                    a = asv[sl] + adv[sl]
                    a = jnp.where(a >= 0.0, a, 0.2 * a)
                    ev[j, sl] = jnp.exp(a - cmaxv)
                pltpu.sync_copy(ev.at[j], s_sh.at[idx_d.at[j]], add=True)
            pltpu.sync_copy(ev, e_hbm.at[pl.ds(gbase, 4)])
            return 0

        lax.fori_loop(0, _RW // 4, group, 0)
        plsc.subcore_barrier()
        stripe = pl.ds(sid * _STRIPE, _STRIPE)

        @pl.when(cid == 0)
        def _():
            pltpu.sync_copy(s_sh.at[stripe], s0_hbm.at[stripe])

        @pl.when(cid == 1)
        def _():
            pltpu.sync_copy(s_sh.at[stripe], s1_hbm.at[stripe])

    return k


def _make_pass2(d):
    nsub = d // 16

    @functools.partial(
        pl.kernel,
        out_type=(
            jax.ShapeDtypeStruct((_NP, d), _F32),
            jax.ShapeDtypeStruct((_NP, d), _F32),
        ),
        mesh=_mesh(),
        compiler_params=pltpu.CompilerParams(use_tc_tiling_on_sc=False),
        scratch_types=(
            pltpu.VMEM((4, 128), _I32),
            pltpu.VMEM((4, 128), _I32),
            pltpu.VMEM((4, 128), _F32),
            pltpu.VMEM((128, d), _F32),
            pltpu.VMEM((128, d), _F32),
            pltpu.VMEM((128,), _F32),
            pltpu.VMEM((128,), _F32),
            pltpu.VMEM((128,), _F32),
            pltpu.VMEM((128,), _F32),
            pltpu.VMEM((128,), _F32),
            pltpu.VMEM_SHARED((_NP, d), _F32),
            pltpu.SemaphoreType.DMA,
            pltpu.SemaphoreType.DMA,
            pltpu.SemaphoreType.DMA,
            pltpu.SemaphoreType.DMA,
        ),
    )
    def k(src_hbm, dst_hbm, e_hbm, s0_hbm, s1_hbm, x_hbm, zin_hbm,
          m0_hbm, m1_hbm,
          idx_s, idx_d, ev, rows0, rows1, sa0, sa1, sb0, sb1, wv, acc,
          sem_g0, sem_g1, sem_s0, sem_s1):
        cid = lax.axis_index("c")
        sid = lax.axis_index("s")
        wid = sid * 2 + cid
        stripe = pl.ds(sid * _STRIPE, _STRIPE)
        pltpu.sync_copy(zin_hbm.at[stripe], acc.at[stripe])
        plsc.subcore_barrier()

        base = wid * _RW
        bufs = [(rows0, sa0, sb0, sem_g0, sem_s0),
                (rows1, sa1, sb1, sem_g1, sem_s1)]

        def fire(j, rows, sa, sb, semg):
            pltpu.async_copy(x_hbm.at[idx_s.at[j]], rows, semg)
            pltpu.async_copy(s0_hbm.at[idx_d.at[j]], sa, semg)
            pltpu.async_copy(s1_hbm.at[idx_d.at[j]], sb, semg)

        def drain_g(j, rows, sa, sb, semg):
            pltpu.make_async_copy(x_hbm.at[idx_s.at[j]], rows, semg).wait()
            pltpu.make_async_copy(s0_hbm.at[idx_d.at[j]], sa, semg).wait()
            pltpu.make_async_copy(s1_hbm.at[idx_d.at[j]], sb, semg).wait()

        def group(g, _):
            gbase = base + g * 4
            pltpu.sync_copy(src_hbm.at[pl.ds(gbase, 4)], idx_s)
            pltpu.sync_copy(dst_hbm.at[pl.ds(gbase, 4)], idx_d)
            pltpu.sync_copy(e_hbm.at[pl.ds(gbase, 4)], ev)
            fire(0, *bufs[0][:4])
            for j in range(4):
                rows, sa, sb, semg, sems = bufs[j % 2]
                orows, osa, osb, osemg, osems = bufs[(j + 1) % 2]
                drain_g(j, rows, sa, sb, semg)
                if j < 3:
                    if j >= 1:
                        pltpu.make_async_copy(
                            orows, acc.at[idx_d.at[j - 1]], osems).wait()
                    fire(j + 1, orows, osa, osb, osemg)
                for t in range(8):
                    sl = pl.ds(t * 16, 16)
                    wv[sl] = ev[j, sl] / (sa[sl] + sb[sl] + 1e-16)

                def scale(b, _):
                    wchunk = wv[pl.ds(b * 16, 16)]
                    for i in range(16):
                        wk = wchunk[i]
                        kk = b * 16 + i
                        for t in range(nsub):
                            sl = pl.ds(t * 16, 16)
                            rows[kk, sl] = rows[kk, sl] * wk
                    return 0

                lax.fori_loop(0, 8, scale, 0)
                pltpu.async_copy(rows, acc.at[idx_d.at[j]], sems, add=True)
            pltpu.make_async_copy(rows0, acc.at[idx_d.at[2]], sem_s0).wait()
            pltpu.make_async_copy(rows1, acc.at[idx_d.at[3]], sem_s1).wait()
            return 0

        lax.fori_loop(0, _RW // 4, group, 0)
        plsc.subcore_barrier()
        stripe = pl.ds(sid * _STRIPE, _STRIPE)

        @pl.when(cid == 0)
        def _():
            pltpu.sync_copy(acc.at[stripe], m0_hbm.at[stripe])

        @pl.when(cid == 1)
        def _():
            pltpu.sync_copy(acc.at[stripe], m1_hbm.at[stripe])

    return k


def _make_head():
    @functools.partial(
        pl.kernel,
        out_type=jax.ShapeDtypeStruct((_LR, 2048), _F32),
        mesh=_mesh(),
        compiler_params=pltpu.CompilerParams(use_tc_tiling_on_sc=False),
        scratch_types=(
            pltpu.VMEM((_LRW, 128), _I32),
            pltpu.VMEM((_LRW, 128), _I32),
            pltpu.VMEM((128, 64), _F32),
            pltpu.VMEM((128, 64), _F32),
            pltpu.VMEM((128, 64), _F32),
            pltpu.VMEM((128, 64), _F32),
            pltpu.VMEM((128, 64), _F32),
            pltpu.VMEM((128, 64), _F32),
            pltpu.VMEM((128, 64), _F32),
            pltpu.VMEM((128, 64), _F32),
            pltpu.VMEM((_LRW, 2048), _F32),
            pltpu.SemaphoreType.DMA,
            pltpu.SemaphoreType.DMA,
            pltpu.SemaphoreType.DMA,
            pltpu.SemaphoreType.DMA,
        ),
    )
    def k(u_hbm, it_hbm, lsrc_hbm, ldst_hbm,
          h_hbm,
          idx_s, idx_d, ub0, ib0, ub1, ib1, ub2, ib2, ub3, ib3, hbuf,
          sem0, sem1, sem2, sem3):
        cid = lax.axis_index("c")
        sid = lax.axis_index("s")
        wid = sid * 2 + cid
        base = wid * _LRW
        pltpu.sync_copy(lsrc_hbm.at[pl.ds(base, _LRW)], idx_s)
        pltpu.sync_copy(ldst_hbm.at[pl.ds(base, _LRW)], idx_d)
        bufs = [(ub0, ib0, sem0), (ub1, ib1, sem1),
                (ub2, ib2, sem2), (ub3, ib3, sem3)]

        def fire(j, ub, ib, sem):
            pltpu.async_copy(u_hbm.at[idx_s.at[j]], ub, sem)
            pltpu.async_copy(it_hbm.at[idx_d.at[j]], ib, sem)

        def drain(j, ub, ib, sem):
            pltpu.make_async_copy(u_hbm.at[idx_s.at[j]], ub, sem).wait()
            pltpu.make_async_copy(it_hbm.at[idx_d.at[j]], ib, sem).wait()

        def compute(j, ub, ib):
            def grp(b, _):
                for i in range(16):
                    kk = b * 16 + i
                    acc = jnp.zeros((16,), _F32)
                    for t in range(4):
                        sl = pl.ds(t * 16, 16)
                        acc = acc + ub[kk, sl] * ib[kk, sl]
                    hbuf[j, pl.ds(b * 256 + i * 16, 16)] = acc
                return 0

            lax.fori_loop(0, 8, grp, 0)

        for r in range(3):
            fire(r, *bufs[r])

        def quad(q, _):
            for r in range(4):
                c = 4 * q + r
                drain(c, *bufs[r])
                @pl.when(c < _LRW - 3)
                def _():
                    fire(c + 3, *bufs[(r + 3) % 4])
                compute(c, bufs[r][0], bufs[r][1])
            return 0

        lax.fori_loop(0, _LRW // 4, quad, 0)
        pltpu.sync_copy(hbuf, h_hbm.at[pl.ds(base, _LRW)])

    return k


def _combine_call(m0, m1):
    R = 2048
    G = _NP // R

    def body(m0_ref, m1_ref, o_ref):
        o_ref[...] = jnp.maximum(m0_ref[...] + m1_ref[...], 0.0)

    return pl.pallas_call(
        body,
        grid=(G,),
        in_specs=[
            pl.BlockSpec((R, 64), lambda i: (i, 0)),
            pl.BlockSpec((R, 64), lambda i: (i, 0)),
        ],
        out_specs=pl.BlockSpec((R, 64), lambda i: (i, 0)),
        out_shape=jax.ShapeDtypeStruct((_NP, 64), _F32),
    )(m0, m1)


def _reduce_head_call(h, wvb):
    R = 128
    G = _LR // R

    def body(h_ref, wvb_ref, o_ref):
        hb = h_ref[...]
        sel = (lax.broadcasted_iota(_I32, (2048, 128), 0) // 16
               == lax.broadcasted_iota(_I32, (2048, 128), 1)).astype(_F32)
        mask = lax.broadcasted_iota(_I32, (128,), 0) == 64
        b0 = jnp.sum(jnp.where(mask, wvb_ref[...], 0.0))
        o_ref[...] = jnp.dot(hb, sel, preferred_element_type=_F32) + b0

    return pl.pallas_call(
        body,
        grid=(G,),
        in_specs=[
            pl.BlockSpec((R, 2048), lambda i: (i, 0)),
            pl.BlockSpec((128,), lambda i: (0,)),
        ],
        out_specs=pl.BlockSpec((R, 128), lambda i: (i, 0)),
        out_shape=jax.ShapeDtypeStruct((_LR, 128), _F32),
    )(h, wvb)


_pass1 = _make_pass1()
_pass2_128 = _make_pass2(128)
_pass2_64 = _make_pass2(64)
_head = _make_head()


def kernel(x_user, x_item, edge_index_ui, edge_index_iu, edge_label_index,
           W1_pu, b1_pu, W1_pi, b1_pi, ls1, ld1, k1_w, k1_b, q1, Wpu1, bpu1,
           W2_pu, b2_pu, W2_pi, b2_pi, ls2, ld2, k2_w, k2_b, q2, Wpu2, bpu2,
           Wpost, bpost):
    xu = jnp.pad(x_user, ((0, _NP - _N), (0, 0)))
    xi = jnp.pad(x_item, ((0, _NP - _N), (0, 0)))
    epad_src = jnp.full((_ER * 128 - _E,), _NP - 1, _I32)
    epad_dst = _N + (jnp.arange(_ER * 128 - _E, dtype=_I32) % (_NP - _N))
    src2d = jnp.concatenate([edge_index_ui[0], epad_src]).reshape(_ER, 128)
    dst2d = jnp.concatenate([edge_index_ui[1], epad_dst]).reshape(_ER, 128)
    lpad = jnp.zeros((_LR * 128 - _L,), _I32)
    lsrc2d = jnp.concatenate([edge_label_index[0], lpad]).reshape(_LR, 128)
    ldst2d = jnp.concatenate([edge_label_index[1], lpad]).reshape(_LR, 128)

    z128 = jnp.zeros((_NP, 128), _F32)
    z64 = jnp.zeros((_NP, 64), _F32)
    pu, as1, ad1, user1, scal1 = _dense1_call(
        xu, xi, W1_pu, b1_pu, W1_pi, b1_pi, ls1, ld1, Wpu1, bpu1)
    e1, s10, s11 = _pass1(src2d, dst2d, as1, ad1, scal1)
    m10, m11 = _pass2_128(src2d, dst2d, e1, s10, s11, pu, z128)
    pu2, as2, ad2, user2, scal2, wvb = _dense2_call(
        m10, m11, user1, W2_pu, b2_pu, W2_pi, b2_pi, ls2, ld2, Wpu2, bpu2,
        Wpost, bpost)
    e2, s20, s21 = _pass1(src2d, dst2d, as2, ad2, scal2)
    m20, m21 = _pass2_64(src2d, dst2d, e2, s20, s21, pu2, z64)
    it2 = _combine_call(m20, m21)
    h = _head(user2, it2, lsrc2d, ldst2d)
    o2d = _reduce_head_call(h, wvb)
    return o2d.reshape(-1)[:_L]


# speedup vs baseline: 1.0268x; 1.0268x over previous
import functools

import jax
import jax.numpy as jnp
from jax import lax
from jax.experimental import pallas as pl
from jax.experimental.pallas import tpu as pltpu
from jax.experimental.pallas import tpu_sc as plsc

_F32 = jnp.float32
_BF16 = jnp.bfloat16
_I32 = jnp.int32
_HMASK = -65536


_N = 10000
_NP = 10240
_E = 320000
_ER = 2560
_NW = 32
_RW = _ER // _NW
_L = 50000
_LR = 512
_LRW = _LR // _NW
_STRIPE = _NP // 16


def _mesh():
    return plsc.VectorSubcoreMesh(core_axis_name="c", subcore_axis_name="s")


def _dense1_call(xu, xi, W1_pu, b1_pu, W1_pi, b1_pi, ls1, ld1, Wpu1, bpu1):
    R = 2048
    G = _NP // R

    def body(xu_ref, xi_ref, wpu_ref, bpu_ref, wpi_ref, bpi_ref, ls_ref,
             ld_ref, wu_ref, bu_ref,
             pu_ref, as_ref, ad_ref, u1_ref, scal_ref):
        i = pl.program_id(0)
        xub = xu_ref[...]
        xib = xi_ref[...]
        pu = jnp.dot(xub, wpu_ref[...], preferred_element_type=_F32)
        pu = pu + bpu_ref[...][None, :]
        pu_ref[...] = pu
        asb = jnp.sum(pu * ls_ref[...][None, :], axis=1)
        as_ref[...] = asb
        pi = jnp.dot(xib, wpi_ref[...], preferred_element_type=_F32)
        pi = pi + bpi_ref[...][None, :]
        adb = jnp.sum(pi * ld_ref[...][None, :], axis=1)
        ad_ref[...] = adb
        u1 = jnp.dot(xub, wu_ref[...], preferred_element_type=_F32)
        u1_ref[...] = jnp.maximum(u1 + bu_ref[...][None, :], 0.0)
        row = jnp.concatenate([
            jnp.full((1, 128), jnp.max(asb), _F32),
            jnp.full((1, 128), jnp.max(adb), _F32),
            jnp.full((6, 128), -1e30, _F32),
        ], axis=0)

        @pl.when(i == 0)
        def _():
            scal_ref[...] = row

        @pl.when(i != 0)
        def _():
            scal_ref[...] = jnp.maximum(scal_ref[...], row)

    def fullspec(s):
        return pl.BlockSpec(s, lambda i: tuple(0 for _ in s))

    return pl.pallas_call(
        body,
        grid=(G,),
        in_specs=[
            pl.BlockSpec((R, 128), lambda i: (i, 0)),
            pl.BlockSpec((R, 128), lambda i: (i, 0)),
            fullspec((128, 128)), fullspec((128,)),
            fullspec((128, 128)), fullspec((128,)),
            fullspec((128,)), fullspec((128,)),
            fullspec((128, 128)), fullspec((128,)),
        ],
        out_specs=[
            pl.BlockSpec((R, 128), lambda i: (i, 0)),
            pl.BlockSpec((R,), lambda i: (i,)),
            pl.BlockSpec((R,), lambda i: (i,)),
            pl.BlockSpec((R, 128), lambda i: (i, 0)),
            pl.BlockSpec((8, 128), lambda i: (0, 0)),
        ],
        out_shape=[
            jax.ShapeDtypeStruct((_NP, 128), _F32),
            jax.ShapeDtypeStruct((_NP,), _F32),
            jax.ShapeDtypeStruct((_NP,), _F32),
            jax.ShapeDtypeStruct((_NP, 128), _F32),
            jax.ShapeDtypeStruct((8, 128), _F32),
        ],
    )(xu, xi, W1_pu, b1_pu, W1_pi, b1_pi, ls1, ld1, Wpu1, bpu1)


def _dense2_call(m0, m1, user1, W2_pu, b2_pu, W2_pi, b2_pi, ls2, ld2,
                 Wpu2, bpu2, Wpost, bpost):
    R = 2048
    G = _NP // R

    def body(m0_ref, m1_ref, u1_ref, wpu_ref, bpu_ref, wpi_ref, bpi_ref,
             ls_ref, ld_ref, wu_ref, bu_ref, wpost_ref, bpost_ref,
             pu2_ref, as_ref, ad_ref, u2_ref, scal_ref, wvb_ref):
        i = pl.program_id(0)
        it1 = jnp.maximum(m0_ref[...] + m1_ref[...], 0.0)
        u1 = u1_ref[...]
        pu2 = jnp.dot(u1, wpu_ref[...], preferred_element_type=_F32)
        pu2 = pu2 + bpu_ref[...][None, :]
        pu2_ref[...] = pu2
        asb = jnp.sum(pu2 * ls_ref[...][None, :], axis=1)
        as_ref[...] = asb
        pi2 = jnp.dot(it1, wpi_ref[...], preferred_element_type=_F32)
        pi2 = pi2 + bpi_ref[...][None, :]
        adb = jnp.sum(pi2 * ld_ref[...][None, :], axis=1)
        ad_ref[...] = adb
        wv = jnp.sum(wpost_ref[...], axis=1)
        b0 = jnp.sum(bpost_ref[...])
        u2 = jnp.dot(u1, wu_ref[...], preferred_element_type=_F32)
        u2 = jnp.maximum(u2 + bu_ref[...][None, :], 0.0)
        u2_ref[...] = u2 * wv[None, :]
        wvb_ref[...] = jnp.concatenate([wv, jnp.full((64,), b0, _F32)], axis=0)
        row = jnp.concatenate([
            jnp.full((1, 128), jnp.max(asb), _F32),
            jnp.full((1, 128), jnp.max(adb), _F32),
            jnp.full((6, 128), -1e30, _F32),
        ], axis=0)

        @pl.when(i == 0)
        def _():
            scal_ref[...] = row

        @pl.when(i != 0)
        def _():
            scal_ref[...] = jnp.maximum(scal_ref[...], row)

    def fullspec(s):
        return pl.BlockSpec(s, lambda i: tuple(0 for _ in s))

    return pl.pallas_call(
        body,
        grid=(G,),
        in_specs=[
            pl.BlockSpec((R, 128), lambda i: (i, 0)),
            pl.BlockSpec((R, 128), lambda i: (i, 0)),
            pl.BlockSpec((R, 128), lambda i: (i, 0)),
            fullspec((128, 64)), fullspec((64,)),
            fullspec((128, 64)), fullspec((64,)),
            fullspec((64,)), fullspec((64,)),
            fullspec((128, 64)), fullspec((64,)),
            fullspec((64, 2)), fullspec((2,)),
        ],
        out_specs=[
            pl.BlockSpec((R, 64), lambda i: (i, 0)),
            pl.BlockSpec((R,), lambda i: (i,)),
            pl.BlockSpec((R,), lambda i: (i,)),
            pl.BlockSpec((R, 64), lambda i: (i, 0)),
            pl.BlockSpec((8, 128), lambda i: (0, 0)),
            pl.BlockSpec((128,), lambda i: (0,)),
        ],
        out_shape=[
            jax.ShapeDtypeStruct((_NP, 64), _F32),
            jax.ShapeDtypeStruct((_NP,), _F32),
            jax.ShapeDtypeStruct((_NP,), _F32),
            jax.ShapeDtypeStruct((_NP, 64), _F32),
            jax.ShapeDtypeStruct((8, 128), _F32),
            jax.ShapeDtypeStruct((128,), _F32),
        ],
    )(m0, m1, user1, W2_pu, b2_pu, W2_pi, b2_pi, ls2, ld2, Wpu2, bpu2,
      Wpost, bpost)


def _make_pass1():
    @functools.partial(
        pl.kernel,
        out_type=(
            jax.ShapeDtypeStruct((_ER, 128), _F32),
            jax.ShapeDtypeStruct((_NP,), _F32),
            jax.ShapeDtypeStruct((_NP,), _F32),
        ),
        mesh=_mesh(),
        compiler_params=pltpu.CompilerParams(use_tc_tiling_on_sc=False),
        scratch_types=(
            pltpu.VMEM((4, 128), _I32),
            pltpu.VMEM((4, 128), _I32),
            pltpu.VMEM((4, 128), _F32),
            pltpu.VMEM((128,), _F32),
            pltpu.VMEM((128,), _F32),
            pltpu.VMEM((128,), _F32),
            pltpu.VMEM((128,), _F32),
            pltpu.VMEM((8, 128), _F32),
            pltpu.VMEM((_STRIPE,), _F32),
            pltpu.VMEM_SHARED((_NP,), _F32),
            pltpu.SemaphoreType.DMA,
            pltpu.SemaphoreType.DMA,
        ),
    )
    def k(src_hbm, dst_hbm, as_hbm, ad_hbm, scal_hbm,
          e_hbm, s0_hbm, s1_hbm,
          idx_s, idx_d, ev, asv0, adv0, asv1, adv1, scalv, zbuf, s_sh,
          sem0, sem1):
        cid = lax.axis_index("c")
        sid = lax.axis_index("s")
        wid = sid * 2 + cid

        def zinit(i, _):
            zbuf[pl.ds(i * 16, 16)] = jnp.zeros((16,), _F32)
            return 0

        lax.fori_loop(0, _STRIPE // 16, zinit, 0)
        pltpu.sync_copy(zbuf, s_sh.at[pl.ds(sid * _STRIPE, _STRIPE)])
        plsc.subcore_barrier()

        base = wid * _RW
        pltpu.sync_copy(scal_hbm, scalv)
        cpre = scalv[0, pl.ds(0, 16)] + scalv[1, pl.ds(0, 16)]
        cmaxv = jnp.where(cpre >= 0.0, cpre, 0.2 * cpre)
        bufs = [(asv0, adv0, sem0), (asv1, adv1, sem1)]

        def fire(j, asv, adv, sem):
            pltpu.async_copy(as_hbm.at[idx_s.at[j]], asv, sem)
            pltpu.async_copy(ad_hbm.at[idx_d.at[j]], adv, sem)

        def drain(j, asv, adv, sem):
            pltpu.make_async_copy(as_hbm.at[idx_s.at[j]], asv, sem).wait()
            pltpu.make_async_copy(ad_hbm.at[idx_d.at[j]], adv, sem).wait()

        def group(g, _):
            gbase = base + g * 4
            pltpu.sync_copy(src_hbm.at[pl.ds(gbase, 4)], idx_s)
            pltpu.sync_copy(dst_hbm.at[pl.ds(gbase, 4)], idx_d)
            fire(0, *bufs[0])
            for j in range(4):
                asv, adv, sem = bufs[j % 2]
                drain(j, asv, adv, sem)
                if j < 3:
                    fire(j + 1, *bufs[(j + 1) % 2])
                for t in range(8):
                    sl = pl.ds(t * 16, 16)
                    a = asv[sl] + adv[sl]
                    a = jnp.where(a >= 0.0, a, 0.2 * a)
                    ev[j, sl] = jnp.exp(a - cmaxv)
                pltpu.sync_copy(ev.at[j], s_sh.at[idx_d.at[j]], add=True)
            pltpu.sync_copy(ev, e_hbm.at[pl.ds(gbase, 4)])
            return 0

        lax.fori_loop(0, _RW // 4, group, 0)
        plsc.subcore_barrier()
        stripe = pl.ds(sid * _STRIPE, _STRIPE)

        @pl.when(cid == 0)
        def _():
            pltpu.sync_copy(s_sh.at[stripe], s0_hbm.at[stripe])

        @pl.when(cid == 1)
        def _():
            pltpu.sync_copy(s_sh.at[stripe], s1_hbm.at[stripe])

    return k


def _make_pass2(d):
    nsub = d // 16

    @functools.partial(
        pl.kernel,
        out_type=(
            jax.ShapeDtypeStruct((_NP, d), _F32),
            jax.ShapeDtypeStruct((_NP, d), _F32),
        ),
        mesh=_mesh(),
        compiler_params=pltpu.CompilerParams(use_tc_tiling_on_sc=False),
        scratch_types=(
            pltpu.VMEM((4, 128), _I32),
            pltpu.VMEM((4, 128), _I32),
            pltpu.VMEM((4, 128), _F32),
            pltpu.VMEM((128, d), _F32),
            pltpu.VMEM((128, d), _F32),
            pltpu.VMEM((128,), _F32),
            pltpu.VMEM((128,), _F32),
            pltpu.VMEM((128,), _F32),
            pltpu.VMEM((128,), _F32),
            pltpu.VMEM((128,), _F32),
            pltpu.VMEM_SHARED((_NP, d), _F32),
            pltpu.SemaphoreType.DMA,
            pltpu.SemaphoreType.DMA,
            pltpu.SemaphoreType.DMA,
            pltpu.SemaphoreType.DMA,
        ),
    )
    def k(src_hbm, dst_hbm, e_hbm, s0_hbm, s1_hbm, x_hbm, zin_hbm,
          m0_hbm, m1_hbm,
          idx_s, idx_d, ev, rows0, rows1, sa0, sa1, sb0, sb1, wv, acc,
          sem_g0, sem_g1, sem_s0, sem_s1):
        cid = lax.axis_index("c")
        sid = lax.axis_index("s")
        wid = sid * 2 + cid
        stripe = pl.ds(sid * _STRIPE, _STRIPE)
        pltpu.sync_copy(zin_hbm.at[stripe], acc.at[stripe])
        plsc.subcore_barrier()

        base = wid * _RW
        bufs = [(rows0, sa0, sb0, sem_g0, sem_s0),
                (rows1, sa1, sb1, sem_g1, sem_s1)]

        def fire(j, rows, sa, sb, semg):
            pltpu.async_copy(x_hbm.at[idx_s.at[j]], rows, semg)
            pltpu.async_copy(s0_hbm.at[idx_d.at[j]], sa, semg)
            pltpu.async_copy(s1_hbm.at[idx_d.at[j]], sb, semg)

        def drain_g(j, rows, sa, sb, semg):
            pltpu.make_async_copy(x_hbm.at[idx_s.at[j]], rows, semg).wait()
            pltpu.make_async_copy(s0_hbm.at[idx_d.at[j]], sa, semg).wait()
            pltpu.make_async_copy(s1_hbm.at[idx_d.at[j]], sb, semg).wait()

        def group(g, _):
            gbase = base + g * 4
            pltpu.sync_copy(src_hbm.at[pl.ds(gbase, 4)], idx_s)
            pltpu.sync_copy(dst_hbm.at[pl.ds(gbase, 4)], idx_d)
            pltpu.sync_copy(e_hbm.at[pl.ds(gbase, 4)], ev)
            fire(0, *bufs[0][:4])
            for j in range(4):
                rows, sa, sb, semg, sems = bufs[j % 2]
                orows, osa, osb, osemg, osems = bufs[(j + 1) % 2]
                drain_g(j, rows, sa, sb, semg)
                if j < 3:
                    if j >= 1:
                        pltpu.make_async_copy(
                            orows, acc.at[idx_d.at[j - 1]], osems).wait()
                    fire(j + 1, orows, osa, osb, osemg)
                for t in range(8):
                    sl = pl.ds(t * 16, 16)
                    wv[sl] = ev[j, sl] / (sa[sl] + sb[sl] + 1e-16)

                def scale(b, _):
                    wchunk = wv[pl.ds(b * 16, 16)]
                    for i in range(16):
                        wk = wchunk[i]
                        kk = b * 16 + i
                        for t in range(nsub):
                            sl = pl.ds(t * 16, 16)
                            rows[kk, sl] = rows[kk, sl] * wk
                    return 0

                lax.fori_loop(0, 8, scale, 0)
                pltpu.async_copy(rows, acc.at[idx_d.at[j]], sems, add=True)
            pltpu.make_async_copy(rows0, acc.at[idx_d.at[2]], sem_s0).wait()
            pltpu.make_async_copy(rows1, acc.at[idx_d.at[3]], sem_s1).wait()
            return 0

        lax.fori_loop(0, _RW // 4, group, 0)
        plsc.subcore_barrier()
        stripe = pl.ds(sid * _STRIPE, _STRIPE)

        @pl.when(cid == 0)
        def _():
            pltpu.sync_copy(acc.at[stripe], m0_hbm.at[stripe])

        @pl.when(cid == 1)
        def _():
            pltpu.sync_copy(acc.at[stripe], m1_hbm.at[stripe])

    return k


def _make_head():
    @functools.partial(
        pl.kernel,
        out_type=jax.ShapeDtypeStruct((_LR, 2048), _F32),
        mesh=_mesh(),
        compiler_params=pltpu.CompilerParams(use_tc_tiling_on_sc=False),
        scratch_types=(
            pltpu.VMEM((_LRW, 128), _I32),
            pltpu.VMEM((_LRW, 128), _I32),
            pltpu.VMEM((128, 64), _F32),
            pltpu.VMEM((128, 64), _F32),
            pltpu.VMEM((128, 64), _F32),
            pltpu.VMEM((128, 64), _F32),
            pltpu.VMEM((_LRW, 2048), _F32),
            pltpu.SemaphoreType.DMA,
            pltpu.SemaphoreType.DMA,
        ),
    )
    def k(u_hbm, it_hbm, lsrc_hbm, ldst_hbm,
          h_hbm,
          idx_s, idx_d, ub0, ib0, ub1, ib1, hbuf,
          sem0, sem1):
        cid = lax.axis_index("c")
        sid = lax.axis_index("s")
        wid = sid * 2 + cid
        base = wid * _LRW
        pltpu.sync_copy(lsrc_hbm.at[pl.ds(base, _LRW)], idx_s)
        pltpu.sync_copy(ldst_hbm.at[pl.ds(base, _LRW)], idx_d)

        def fire(j, ub, ib, sem):
            pltpu.async_copy(u_hbm.at[idx_s.at[j]], ub, sem)
            pltpu.async_copy(it_hbm.at[idx_d.at[j]], ib, sem)

        def drain(j, ub, ib, sem):
            pltpu.make_async_copy(u_hbm.at[idx_s.at[j]], ub, sem).wait()
            pltpu.make_async_copy(it_hbm.at[idx_d.at[j]], ib, sem).wait()

        def compute(j, ub, ib):
            def grp(b, _):
                for i in range(16):
                    kk = b * 16 + i
                    acc = jnp.zeros((16,), _F32)
                    for t in range(4):
                        sl = pl.ds(t * 16, 16)
                        acc = acc + ub[kk, sl] * ib[kk, sl]
                    hbuf[j, pl.ds(b * 256 + i * 16, 16)] = acc
                return 0

            lax.fori_loop(0, 8, grp, 0)

        fire(0, ub0, ib0, sem0)

        def pair(jj, _):
            j0 = 2 * jj
            drain(j0, ub0, ib0, sem0)
            fire(j0 + 1, ub1, ib1, sem1)
            compute(j0, ub0, ib0)
            drain(j0 + 1, ub1, ib1, sem1)
            fire(j0 + 2, ub0, ib0, sem0)
            compute(j0 + 1, ub1, ib1)
            return 0

        lax.fori_loop(0, (_LRW - 2) // 2, pair, 0)
        j0 = _LRW - 2
        drain(j0, ub0, ib0, sem0)
        fire(j0 + 1, ub1, ib1, sem1)
        compute(j0, ub0, ib0)
        drain(j0 + 1, ub1, ib1, sem1)
        compute(j0 + 1, ub1, ib1)
        pltpu.sync_copy(hbuf, h_hbm.at[pl.ds(base, _LRW)])

    return k


def _combine_call(m0, m1):
    R = 2048
    G = _NP // R

    def body(m0_ref, m1_ref, o_ref):
        o_ref[...] = jnp.maximum(m0_ref[...] + m1_ref[...], 0.0)

    return pl.pallas_call(
        body,
        grid=(G,),
        in_specs=[
            pl.BlockSpec((R, 64), lambda i: (i, 0)),
            pl.BlockSpec((R, 64), lambda i: (i, 0)),
        ],
        out_specs=pl.BlockSpec((R, 64), lambda i: (i, 0)),
        out_shape=jax.ShapeDtypeStruct((_NP, 64), _F32),
    )(m0, m1)


def _reduce_head_call(h, wvb):
    R = 128
    G = _LR // R

    def body(h_ref, wvb_ref, o_ref):
        hb = h_ref[...]
        sel = (lax.broadcasted_iota(_I32, (2048, 128), 0) // 16
               == lax.broadcasted_iota(_I32, (2048, 128), 1)).astype(_F32)
        mask = lax.broadcasted_iota(_I32, (128,), 0) == 64
        b0 = jnp.sum(jnp.where(mask, wvb_ref[...], 0.0))
        o_ref[...] = jnp.dot(hb, sel, preferred_element_type=_F32) + b0

    return pl.pallas_call(
        body,
        grid=(G,),
        in_specs=[
            pl.BlockSpec((R, 2048), lambda i: (i, 0)),
            pl.BlockSpec((128,), lambda i: (0,)),
        ],
        out_specs=pl.BlockSpec((R, 128), lambda i: (i, 0)),
        out_shape=jax.ShapeDtypeStruct((_LR, 128), _F32),
    )(h, wvb)


_pass1 = _make_pass1()
_pass2_128 = _make_pass2(128)
_pass2_64 = _make_pass2(64)
_head = _make_head()


def kernel(x_user, x_item, edge_index_ui, edge_index_iu, edge_label_index,
           W1_pu, b1_pu, W1_pi, b1_pi, ls1, ld1, k1_w, k1_b, q1, Wpu1, bpu1,
           W2_pu, b2_pu, W2_pi, b2_pi, ls2, ld2, k2_w, k2_b, q2, Wpu2, bpu2,
           Wpost, bpost):
    xu = jnp.pad(x_user, ((0, _NP - _N), (0, 0)))
    xi = jnp.pad(x_item, ((0, _NP - _N), (0, 0)))
    epad_src = jnp.full((_ER * 128 - _E,), _NP - 1, _I32)
    epad_dst = _N + (jnp.arange(_ER * 128 - _E, dtype=_I32) % (_NP - _N))
    src2d = jnp.concatenate([edge_index_ui[0], epad_src]).reshape(_ER, 128)
    dst2d = jnp.concatenate([edge_index_ui[1], epad_dst]).reshape(_ER, 128)
    lpad = jnp.zeros((_LR * 128 - _L,), _I32)
    lsrc2d = jnp.concatenate([edge_label_index[0], lpad]).reshape(_LR, 128)
    ldst2d = jnp.concatenate([edge_label_index[1], lpad]).reshape(_LR, 128)

    z128 = jnp.zeros((_NP, 128), _F32)
    z64 = jnp.zeros((_NP, 64), _F32)
    pu, as1, ad1, user1, scal1 = _dense1_call(
        xu, xi, W1_pu, b1_pu, W1_pi, b1_pi, ls1, ld1, Wpu1, bpu1)
    e1, s10, s11 = _pass1(src2d, dst2d, as1, ad1, scal1)
    m10, m11 = _pass2_128(src2d, dst2d, e1, s10, s11, pu, z128)
    pu2, as2, ad2, user2, scal2, wvb = _dense2_call(
        m10, m11, user1, W2_pu, b2_pu, W2_pi, b2_pi, ls2, ld2, Wpu2, bpu2,
        Wpost, bpost)
    e2, s20, s21 = _pass1(src2d, dst2d, as2, ad2, scal2)
    m20, m21 = _pass2_64(src2d, dst2d, e2, s20, s21, pu2, z64)
    it2 = _combine_call(m20, m21)
    h = _head(user2, it2, lsrc2d, ldst2d)
    o2d = _reduce_head_call(h, wvb)
    return o2d.reshape(-1)[:_L]


# speedup vs baseline: 1.2149x; 1.1831x over previous
import functools

import jax
import jax.numpy as jnp
from jax import lax
from jax.experimental import pallas as pl
from jax.experimental.pallas import tpu as pltpu
from jax.experimental.pallas import tpu_sc as plsc

_F32 = jnp.float32
_BF16 = jnp.bfloat16
_I32 = jnp.int32
_HMASK = -65536


_N = 10000
_NP = 10240
_E = 320000
_ER = 2560
_NW = 32
_RW = _ER // _NW
_L = 50000
_LR = 416
_LRW = _LR // _NW
_STRIPE = _NP // 16


def _mesh():
    return plsc.VectorSubcoreMesh(core_axis_name="c", subcore_axis_name="s")


def _dense1_call(xu, xi, W1_pu, b1_pu, W1_pi, b1_pi, ls1, ld1, Wpu1, bpu1):
    R = 2048
    G = _NP // R

    def body(xu_ref, xi_ref, wpu_ref, bpu_ref, wpi_ref, bpi_ref, ls_ref,
             ld_ref, wu_ref, bu_ref,
             pu_ref, as_ref, ad_ref, u1_ref, scal_ref):
        i = pl.program_id(0)
        xub = xu_ref[...]
        xib = xi_ref[...]
        pu = jnp.dot(xub, wpu_ref[...], preferred_element_type=_F32)
        pu = pu + bpu_ref[...][None, :]
        pu_ref[...] = pu
        asb = jnp.sum(pu * ls_ref[...][None, :], axis=1)
        as_ref[...] = asb
        pi = jnp.dot(xib, wpi_ref[...], preferred_element_type=_F32)
        pi = pi + bpi_ref[...][None, :]
        adb = jnp.sum(pi * ld_ref[...][None, :], axis=1)
        ad_ref[...] = adb
        u1 = jnp.dot(xub, wu_ref[...], preferred_element_type=_F32)
        u1_ref[...] = jnp.maximum(u1 + bu_ref[...][None, :], 0.0)
        row = jnp.concatenate([
            jnp.full((1, 128), jnp.max(asb), _F32),
            jnp.full((1, 128), jnp.max(adb), _F32),
            jnp.full((6, 128), -1e30, _F32),
        ], axis=0)

        @pl.when(i == 0)
        def _():
            scal_ref[...] = row

        @pl.when(i != 0)
        def _():
            scal_ref[...] = jnp.maximum(scal_ref[...], row)

    def fullspec(s):
        return pl.BlockSpec(s, lambda i: tuple(0 for _ in s))

    return pl.pallas_call(
        body,
        grid=(G,),
        in_specs=[
            pl.BlockSpec((R, 128), lambda i: (i, 0)),
            pl.BlockSpec((R, 128), lambda i: (i, 0)),
            fullspec((128, 128)), fullspec((128,)),
            fullspec((128, 128)), fullspec((128,)),
            fullspec((128,)), fullspec((128,)),
            fullspec((128, 128)), fullspec((128,)),
        ],
        out_specs=[
            pl.BlockSpec((R, 128), lambda i: (i, 0)),
            pl.BlockSpec((R,), lambda i: (i,)),
            pl.BlockSpec((R,), lambda i: (i,)),
            pl.BlockSpec((R, 128), lambda i: (i, 0)),
            pl.BlockSpec((8, 128), lambda i: (0, 0)),
        ],
        out_shape=[
            jax.ShapeDtypeStruct((_NP, 128), _F32),
            jax.ShapeDtypeStruct((_NP,), _F32),
            jax.ShapeDtypeStruct((_NP,), _F32),
            jax.ShapeDtypeStruct((_NP, 128), _F32),
            jax.ShapeDtypeStruct((8, 128), _F32),
        ],
    )(xu, xi, W1_pu, b1_pu, W1_pi, b1_pi, ls1, ld1, Wpu1, bpu1)


def _dense2_call(m0, m1, user1, W2_pu, b2_pu, W2_pi, b2_pi, ls2, ld2,
                 Wpu2, bpu2, Wpost, bpost):
    R = 2048
    G = _NP // R

    def body(m0_ref, m1_ref, u1_ref, wpu_ref, bpu_ref, wpi_ref, bpi_ref,
             ls_ref, ld_ref, wu_ref, bu_ref, wpost_ref, bpost_ref,
             pu2_ref, as_ref, ad_ref, u2_ref, scal_ref, wvb_ref):
        i = pl.program_id(0)
        it1 = jnp.maximum(m0_ref[...] + m1_ref[...], 0.0)
        u1 = u1_ref[...]
        pu2 = jnp.dot(u1, wpu_ref[...], preferred_element_type=_F32)
        pu2 = pu2 + bpu_ref[...][None, :]
        pu2_ref[...] = pu2
        asb = jnp.sum(pu2 * ls_ref[...][None, :], axis=1)
        as_ref[...] = asb
        pi2 = jnp.dot(it1, wpi_ref[...], preferred_element_type=_F32)
        pi2 = pi2 + bpi_ref[...][None, :]
        adb = jnp.sum(pi2 * ld_ref[...][None, :], axis=1)
        ad_ref[...] = adb
        wv = jnp.sum(wpost_ref[...], axis=1)
        b0 = jnp.sum(bpost_ref[...])
        u2 = jnp.dot(u1, wu_ref[...], preferred_element_type=_F32)
        u2 = jnp.maximum(u2 + bu_ref[...][None, :], 0.0)
        u2_ref[...] = u2 * wv[None, :]
        wvb_ref[...] = jnp.concatenate([wv, jnp.full((64,), b0, _F32)], axis=0)
        row = jnp.concatenate([
            jnp.full((1, 128), jnp.max(asb), _F32),
            jnp.full((1, 128), jnp.max(adb), _F32),
            jnp.full((6, 128), -1e30, _F32),
        ], axis=0)

        @pl.when(i == 0)
        def _():
            scal_ref[...] = row

        @pl.when(i != 0)
        def _():
            scal_ref[...] = jnp.maximum(scal_ref[...], row)

    def fullspec(s):
        return pl.BlockSpec(s, lambda i: tuple(0 for _ in s))

    return pl.pallas_call(
        body,
        grid=(G,),
        in_specs=[
            pl.BlockSpec((R, 128), lambda i: (i, 0)),
            pl.BlockSpec((R, 128), lambda i: (i, 0)),
            pl.BlockSpec((R, 128), lambda i: (i, 0)),
            fullspec((128, 64)), fullspec((64,)),
            fullspec((128, 64)), fullspec((64,)),
            fullspec((64,)), fullspec((64,)),
            fullspec((128, 64)), fullspec((64,)),
            fullspec((64, 2)), fullspec((2,)),
        ],
        out_specs=[
            pl.BlockSpec((R, 64), lambda i: (i, 0)),
            pl.BlockSpec((R,), lambda i: (i,)),
            pl.BlockSpec((R,), lambda i: (i,)),
            pl.BlockSpec((R, 64), lambda i: (i, 0)),
            pl.BlockSpec((8, 128), lambda i: (0, 0)),
            pl.BlockSpec((128,), lambda i: (0,)),
        ],
        out_shape=[
            jax.ShapeDtypeStruct((_NP, 64), _F32),
            jax.ShapeDtypeStruct((_NP,), _F32),
            jax.ShapeDtypeStruct((_NP,), _F32),
            jax.ShapeDtypeStruct((_NP, 64), _F32),
            jax.ShapeDtypeStruct((8, 128), _F32),
            jax.ShapeDtypeStruct((128,), _F32),
        ],
    )(m0, m1, user1, W2_pu, b2_pu, W2_pi, b2_pi, ls2, ld2, Wpu2, bpu2,
      Wpost, bpost)


def _make_pass1():
    @functools.partial(
        pl.kernel,
        out_type=(
            jax.ShapeDtypeStruct((_ER, 128), _F32),
            jax.ShapeDtypeStruct((_NP,), _F32),
            jax.ShapeDtypeStruct((_NP,), _F32),
        ),
        mesh=_mesh(),
        compiler_params=pltpu.CompilerParams(use_tc_tiling_on_sc=False),
        scratch_types=(
            pltpu.VMEM((4, 128), _I32),
            pltpu.VMEM((4, 128), _I32),
            pltpu.VMEM((4, 128), _F32),
            pltpu.VMEM((128,), _F32),
            pltpu.VMEM((128,), _F32),
            pltpu.VMEM((128,), _F32),
            pltpu.VMEM((128,), _F32),
            pltpu.VMEM((8, 128), _F32),
            pltpu.VMEM((_STRIPE,), _F32),
            pltpu.VMEM_SHARED((_NP,), _F32),
            pltpu.SemaphoreType.DMA,
            pltpu.SemaphoreType.DMA,
        ),
    )
    def k(src_hbm, dst_hbm, as_hbm, ad_hbm, scal_hbm,
          e_hbm, s0_hbm, s1_hbm,
          idx_s, idx_d, ev, asv0, adv0, asv1, adv1, scalv, zbuf, s_sh,
          sem0, sem1):
        cid = lax.axis_index("c")
        sid = lax.axis_index("s")
        wid = sid * 2 + cid

        def zinit(i, _):
            zbuf[pl.ds(i * 16, 16)] = jnp.zeros((16,), _F32)
            return 0

        lax.fori_loop(0, _STRIPE // 16, zinit, 0)
        pltpu.sync_copy(zbuf, s_sh.at[pl.ds(sid * _STRIPE, _STRIPE)])
        plsc.subcore_barrier()

        base = wid * _RW
        pltpu.sync_copy(scal_hbm, scalv)
        cpre = scalv[0, pl.ds(0, 16)] + scalv[1, pl.ds(0, 16)]
        cmaxv = jnp.where(cpre >= 0.0, cpre, 0.2 * cpre)
        bufs = [(asv0, adv0, sem0), (asv1, adv1, sem1)]

        def fire(j, asv, adv, sem):
            pltpu.async_copy(as_hbm.at[idx_s.at[j]], asv, sem)
            pltpu.async_copy(ad_hbm.at[idx_d.at[j]], adv, sem)

        def drain(j, asv, adv, sem):
            pltpu.make_async_copy(as_hbm.at[idx_s.at[j]], asv, sem).wait()
            pltpu.make_async_copy(ad_hbm.at[idx_d.at[j]], adv, sem).wait()

        def group(g, _):
            gbase = base + g * 4
            pltpu.sync_copy(src_hbm.at[pl.ds(gbase, 4)], idx_s)
            pltpu.sync_copy(dst_hbm.at[pl.ds(gbase, 4)], idx_d)
            fire(0, *bufs[0])
            for j in range(4):
                asv, adv, sem = bufs[j % 2]
                drain(j, asv, adv, sem)
                if j < 3:
                    fire(j + 1, *bufs[(j + 1) % 2])
                for t in range(8):
                    sl = pl.ds(t * 16, 16)
                    a = asv[sl] + adv[sl]
                    a = jnp.where(a >= 0.0, a, 0.2 * a)
                    ev[j, sl] = jnp.exp(a - cmaxv)
                pltpu.sync_copy(ev.at[j], s_sh.at[idx_d.at[j]], add=True)
            pltpu.sync_copy(ev, e_hbm.at[pl.ds(gbase, 4)])
            return 0

        lax.fori_loop(0, _RW // 4, group, 0)
        plsc.subcore_barrier()
        stripe = pl.ds(sid * _STRIPE, _STRIPE)

        @pl.when(cid == 0)
        def _():
            pltpu.sync_copy(s_sh.at[stripe], s0_hbm.at[stripe])

        @pl.when(cid == 1)
        def _():
            pltpu.sync_copy(s_sh.at[stripe], s1_hbm.at[stripe])

    return k


def _make_pass2(d):
    nsub = d // 16

    @functools.partial(
        pl.kernel,
        out_type=(
            jax.ShapeDtypeStruct((_NP, d), _F32),
            jax.ShapeDtypeStruct((_NP, d), _F32),
        ),
        mesh=_mesh(),
        compiler_params=pltpu.CompilerParams(use_tc_tiling_on_sc=False),
        scratch_types=(
            pltpu.VMEM((4, 128), _I32),
            pltpu.VMEM((4, 128), _I32),
            pltpu.VMEM((4, 128), _F32),
            pltpu.VMEM((128, d), _F32),
            pltpu.VMEM((128, d), _F32),
            pltpu.VMEM((128,), _F32),
            pltpu.VMEM((128,), _F32),
            pltpu.VMEM((128,), _F32),
            pltpu.VMEM((128,), _F32),
            pltpu.VMEM((128,), _F32),
            pltpu.VMEM_SHARED((_NP, d), _F32),
            pltpu.SemaphoreType.DMA,
            pltpu.SemaphoreType.DMA,
            pltpu.SemaphoreType.DMA,
            pltpu.SemaphoreType.DMA,
        ),
    )
    def k(src_hbm, dst_hbm, e_hbm, s0_hbm, s1_hbm, x_hbm, zin_hbm,
          m0_hbm, m1_hbm,
          idx_s, idx_d, ev, rows0, rows1, sa0, sa1, sb0, sb1, wv, acc,
          sem_g0, sem_g1, sem_s0, sem_s1):
        cid = lax.axis_index("c")
        sid = lax.axis_index("s")
        wid = sid * 2 + cid
        stripe = pl.ds(sid * _STRIPE, _STRIPE)
        pltpu.sync_copy(zin_hbm.at[stripe], acc.at[stripe])
        plsc.subcore_barrier()

        base = wid * _RW
        bufs = [(rows0, sa0, sb0, sem_g0, sem_s0),
                (rows1, sa1, sb1, sem_g1, sem_s1)]

        def fire(j, rows, sa, sb, semg):
            pltpu.async_copy(x_hbm.at[idx_s.at[j]], rows, semg)
            pltpu.async_copy(s0_hbm.at[idx_d.at[j]], sa, semg)
            pltpu.async_copy(s1_hbm.at[idx_d.at[j]], sb, semg)

        def drain_g(j, rows, sa, sb, semg):
            pltpu.make_async_copy(x_hbm.at[idx_s.at[j]], rows, semg).wait()
            pltpu.make_async_copy(s0_hbm.at[idx_d.at[j]], sa, semg).wait()
            pltpu.make_async_copy(s1_hbm.at[idx_d.at[j]], sb, semg).wait()

        def group(g, _):
            gbase = base + g * 4
            pltpu.sync_copy(src_hbm.at[pl.ds(gbase, 4)], idx_s)
            pltpu.sync_copy(dst_hbm.at[pl.ds(gbase, 4)], idx_d)
            pltpu.sync_copy(e_hbm.at[pl.ds(gbase, 4)], ev)
            fire(0, *bufs[0][:4])
            for j in range(4):
                rows, sa, sb, semg, sems = bufs[j % 2]
                orows, osa, osb, osemg, osems = bufs[(j + 1) % 2]
                drain_g(j, rows, sa, sb, semg)
                if j < 3:
                    if j >= 1:
                        pltpu.make_async_copy(
                            orows, acc.at[idx_d.at[j - 1]], osems).wait()
                    fire(j + 1, orows, osa, osb, osemg)
                for t in range(8):
                    sl = pl.ds(t * 16, 16)
                    wv[sl] = ev[j, sl] / (sa[sl] + sb[sl] + 1e-16)

                def scale(b, _):
                    wchunk = wv[pl.ds(b * 16, 16)]
                    for i in range(16):
                        wk = wchunk[i]
                        kk = b * 16 + i
                        for t in range(nsub):
                            sl = pl.ds(t * 16, 16)
                            rows[kk, sl] = rows[kk, sl] * wk
                    return 0

                lax.fori_loop(0, 8, scale, 0)
                pltpu.async_copy(rows, acc.at[idx_d.at[j]], sems, add=True)
            pltpu.make_async_copy(rows0, acc.at[idx_d.at[2]], sem_s0).wait()
            pltpu.make_async_copy(rows1, acc.at[idx_d.at[3]], sem_s1).wait()
            return 0

        lax.fori_loop(0, _RW // 4, group, 0)
        plsc.subcore_barrier()
        stripe = pl.ds(sid * _STRIPE, _STRIPE)

        @pl.when(cid == 0)
        def _():
            pltpu.sync_copy(acc.at[stripe], m0_hbm.at[stripe])

        @pl.when(cid == 1)
        def _():
            pltpu.sync_copy(acc.at[stripe], m1_hbm.at[stripe])

    return k


def _make_head():
    @functools.partial(
        pl.kernel,
        out_type=jax.ShapeDtypeStruct((_LR, 2048), _F32),
        mesh=_mesh(),
        compiler_params=pltpu.CompilerParams(use_tc_tiling_on_sc=False),
        scratch_types=(
            pltpu.VMEM((_LRW, 128), _I32),
            pltpu.VMEM((_LRW, 128), _I32),
            pltpu.VMEM((128, 64), _F32),
            pltpu.VMEM((128, 64), _F32),
            pltpu.VMEM((128, 64), _F32),
            pltpu.VMEM((128, 64), _F32),
            pltpu.VMEM((_LRW, 2048), _F32),
            pltpu.SemaphoreType.DMA,
            pltpu.SemaphoreType.DMA,
        ),
    )
    def k(u_hbm, it_hbm, lsrc_hbm, ldst_hbm,
          h_hbm,
          idx_s, idx_d, ub0, ib0, ub1, ib1, hbuf,
          sem0, sem1):
        cid = lax.axis_index("c")
        sid = lax.axis_index("s")
        wid = sid * 2 + cid
        base = wid * _LRW
        pltpu.sync_copy(lsrc_hbm.at[pl.ds(base, _LRW)], idx_s)
        pltpu.sync_copy(ldst_hbm.at[pl.ds(base, _LRW)], idx_d)

        def fire(j, ub, ib, sem):
            pltpu.async_copy(u_hbm.at[idx_s.at[j]], ub, sem)
            pltpu.async_copy(it_hbm.at[idx_d.at[j]], ib, sem)

        def drain(j, ub, ib, sem):
            pltpu.make_async_copy(u_hbm.at[idx_s.at[j]], ub, sem).wait()
            pltpu.make_async_copy(it_hbm.at[idx_d.at[j]], ib, sem).wait()

        def compute(j, ub, ib):
            def grp(b, _):
                for i in range(16):
                    kk = b * 16 + i
                    acc = jnp.zeros((16,), _F32)
                    for t in range(4):
                        sl = pl.ds(t * 16, 16)
                        acc = acc + ub[kk, sl] * ib[kk, sl]
                    hbuf[j, pl.ds(b * 256 + i * 16, 16)] = acc
                return 0

            lax.fori_loop(0, 8, grp, 0)

        fire(0, ub0, ib0, sem0)

        def pair(jj, _):
            j0 = 2 * jj
            drain(j0, ub0, ib0, sem0)
            fire(j0 + 1, ub1, ib1, sem1)
            compute(j0, ub0, ib0)
            drain(j0 + 1, ub1, ib1, sem1)
            fire(j0 + 2, ub0, ib0, sem0)
            compute(j0 + 1, ub1, ib1)
            return 0

        lax.fori_loop(0, (_LRW - 1) // 2, pair, 0)
        j0 = _LRW - 1
        drain(j0, ub0, ib0, sem0)
        compute(j0, ub0, ib0)
        pltpu.sync_copy(hbuf, h_hbm.at[pl.ds(base, _LRW)])

    return k


def _combine_call(m0, m1):
    R = 2048
    G = _NP // R

    def body(m0_ref, m1_ref, o_ref):
        o_ref[...] = jnp.maximum(m0_ref[...] + m1_ref[...], 0.0)

    return pl.pallas_call(
        body,
        grid=(G,),
        in_specs=[
            pl.BlockSpec((R, 64), lambda i: (i, 0)),
            pl.BlockSpec((R, 64), lambda i: (i, 0)),
        ],
        out_specs=pl.BlockSpec((R, 64), lambda i: (i, 0)),
        out_shape=jax.ShapeDtypeStruct((_NP, 64), _F32),
    )(m0, m1)


def _reduce_head_call(h, wvb):
    R = 104
    G = _LR // R

    def body(h_ref, wvb_ref, o_ref):
        hb = h_ref[...]
        sel = (lax.broadcasted_iota(_I32, (2048, 128), 0) // 16
               == lax.broadcasted_iota(_I32, (2048, 128), 1)).astype(_F32)
        mask = lax.broadcasted_iota(_I32, (128,), 0) == 64
        b0 = jnp.sum(jnp.where(mask, wvb_ref[...], 0.0))
        o_ref[...] = jnp.dot(hb, sel, preferred_element_type=_F32) + b0

    return pl.pallas_call(
        body,
        grid=(G,),
        in_specs=[
            pl.BlockSpec((R, 2048), lambda i: (i, 0)),
            pl.BlockSpec((128,), lambda i: (0,)),
        ],
        out_specs=pl.BlockSpec((R, 128), lambda i: (i, 0)),
        out_shape=jax.ShapeDtypeStruct((_LR, 128), _F32),
    )(h, wvb)


_pass1 = _make_pass1()
_pass2_128 = _make_pass2(128)
_pass2_64 = _make_pass2(64)
_head = _make_head()


def kernel(x_user, x_item, edge_index_ui, edge_index_iu, edge_label_index,
           W1_pu, b1_pu, W1_pi, b1_pi, ls1, ld1, k1_w, k1_b, q1, Wpu1, bpu1,
           W2_pu, b2_pu, W2_pi, b2_pi, ls2, ld2, k2_w, k2_b, q2, Wpu2, bpu2,
           Wpost, bpost):
    xu = jnp.pad(x_user, ((0, _NP - _N), (0, 0)))
    xi = jnp.pad(x_item, ((0, _NP - _N), (0, 0)))
    epad_src = jnp.full((_ER * 128 - _E,), _NP - 1, _I32)
    epad_dst = _N + (jnp.arange(_ER * 128 - _E, dtype=_I32) % (_NP - _N))
    src2d = jnp.concatenate([edge_index_ui[0], epad_src]).reshape(_ER, 128)
    dst2d = jnp.concatenate([edge_index_ui[1], epad_dst]).reshape(_ER, 128)
    lpad = jnp.zeros((_LR * 128 - _L,), _I32)
    lsrc2d = jnp.concatenate([edge_label_index[0], lpad]).reshape(_LR, 128)
    ldst2d = jnp.concatenate([edge_label_index[1], lpad]).reshape(_LR, 128)

    z128 = jnp.zeros((_NP, 128), _F32)
    z64 = jnp.zeros((_NP, 64), _F32)
    pu, as1, ad1, user1, scal1 = _dense1_call(
        xu, xi, W1_pu, b1_pu, W1_pi, b1_pi, ls1, ld1, Wpu1, bpu1)
    e1, s10, s11 = _pass1(src2d, dst2d, as1, ad1, scal1)
    m10, m11 = _pass2_128(src2d, dst2d, e1, s10, s11, pu, z128)
    pu2, as2, ad2, user2, scal2, wvb = _dense2_call(
        m10, m11, user1, W2_pu, b2_pu, W2_pi, b2_pi, ls2, ld2, Wpu2, bpu2,
        Wpost, bpost)
    e2, s20, s21 = _pass1(src2d, dst2d, as2, ad2, scal2)
    m20, m21 = _pass2_64(src2d, dst2d, e2, s20, s21, pu2, z64)
    it2 = _combine_call(m20, m21)
    h = _head(user2, it2, lsrc2d, ldst2d)
    o2d = _reduce_head_call(h, wvb)
    return o2d.reshape(-1)[:_L]
